# initial kernel scaffold (unmeasured)
import jax
import jax.numpy as jnp
from jax import lax
from jax.experimental import pallas as pl
from jax.experimental.pallas import tpu as pltpu


def kernel(
    x,
):
    def body(*refs):
        pass

    out_shape = jax.ShapeDtypeStruct(..., jnp.float32)
    return pl.pallas_call(body, out_shape=out_shape)(...)


# baseline (device time: 11812 ns/iter reference)
import jax
import jax.numpy as jnp
from jax import lax
from jax.experimental import pallas as pl
from jax.experimental.pallas import tpu as pltpu


def kernel(x):
    m, n = x.shape

    def body(x_ref, out_ref, partial_ref, recv_ref, send_sem, recv_sem):
        my_x = lax.axis_index("x")
        my_y = lax.axis_index("y")
        nbr = (my_x, 1 - my_y)

        barrier_sem = pltpu.get_barrier_semaphore()
        pl.semaphore_signal(
            barrier_sem, inc=1, device_id=nbr,
            device_id_type=pl.DeviceIdType.MESH,
        )
        pl.semaphore_wait(barrier_sem, 1)

        partial_ref[:, :] = jnp.sum(x_ref[:, :], axis=1, keepdims=True)

        rdma = pltpu.make_async_remote_copy(
            src_ref=partial_ref,
            dst_ref=recv_ref,
            send_sem=send_sem,
            recv_sem=recv_sem,
            device_id=nbr,
            device_id_type=pl.DeviceIdType.MESH,
        )
        rdma.start()
        rdma.wait()

        out_ref[:, :] = partial_ref[:, :] + recv_ref[:, :]

    return pl.pallas_call(
        body,
        out_shape=jax.ShapeDtypeStruct((m, 1), jnp.float32),
        in_specs=[pl.BlockSpec(memory_space=pltpu.VMEM)],
        out_specs=pl.BlockSpec(memory_space=pltpu.VMEM),
        scratch_shapes=[
            pltpu.VMEM((m, 1), jnp.float32),
            pltpu.VMEM((m, 1), jnp.float32),
            pltpu.SemaphoreType.DMA,
            pltpu.SemaphoreType.DMA,
        ],
        compiler_params=pltpu.CompilerParams(collective_id=0),
    )(x)


# device time: 5961 ns/iter; 1.9815x vs baseline; 1.9815x over previous
import jax
import jax.numpy as jnp
from jax import lax
from jax.experimental import pallas as pl
from jax.experimental.pallas import tpu as pltpu


def kernel(x):
    m, n = x.shape
    sub = m // 128
    half = sub // 2
    mh = m // 2

    def body(
        x_hbm,
        out_hbm,
        buf,
        send_ref,
        recv_ref,
        comb_ref,
        copy_sems,
        send_sems,
        recv_sems,
        out_sem,
    ):
        my_x = lax.axis_index("x")
        my_y = lax.axis_index("y")
        nbr = (my_x, 1 - my_y)

        barrier_sem = pltpu.get_barrier_semaphore()
        pl.semaphore_signal(
            barrier_sem, inc=1, device_id=nbr,
            device_id_type=pl.DeviceIdType.MESH,
        )

        cp0 = pltpu.make_async_copy(
            x_hbm.at[pl.ds(0, mh), :], buf.at[0], copy_sems.at[0]
        )
        cp1 = pltpu.make_async_copy(
            x_hbm.at[pl.ds(mh, mh), :], buf.at[1], copy_sems.at[1]
        )
        cp0.start()
        cp1.start()

        def reduce_half(b):
            s = (buf[b, :, 0:128] + buf[b, :, 128:256]) + (
                buf[b, :, 256:384] + buf[b, :, 384:512]
            )
            return jnp.sum(s.reshape(half, 128, 128), axis=2)

        def exchange(t, sem_idx):
            return pltpu.make_async_remote_copy(
                src_ref=send_ref.at[pl.ds(t, half)],
                dst_ref=recv_ref.at[pl.ds(t, half)],
                send_sem=send_sems.at[sem_idx],
                recv_sem=recv_sems.at[sem_idx],
                device_id=nbr,
                device_id_type=pl.DeviceIdType.MESH,
            )

        cp0.wait()
        send_ref[0:half, :] = reduce_half(0)
        pl.semaphore_wait(barrier_sem, 1)
        rdma0 = exchange(0, 0)
        rdma0.start()

        cp1.wait()
        send_ref[half:sub, :] = reduce_half(1)
        rdma1 = exchange(half, 1)
        rdma1.start()

        rdma0.wait()
        rdma1.wait()
        comb_ref[:, :] = send_ref[:, :] + recv_ref[:, :]
        out_cp = pltpu.make_async_copy(comb_ref, out_hbm, out_sem)
        out_cp.start()
        out_cp.wait()

    out = pl.pallas_call(
        body,
        out_shape=jax.ShapeDtypeStruct((sub, 128), jnp.float32),
        in_specs=[pl.BlockSpec(memory_space=pltpu.MemorySpace.HBM)],
        out_specs=pl.BlockSpec(memory_space=pltpu.MemorySpace.HBM),
        scratch_shapes=[
            pltpu.VMEM((2, mh, n), jnp.float32),
            pltpu.VMEM((sub, 128), jnp.float32),
            pltpu.VMEM((sub, 128), jnp.float32),
            pltpu.VMEM((sub, 128), jnp.float32),
            pltpu.SemaphoreType.DMA((2,)),
            pltpu.SemaphoreType.DMA((2,)),
            pltpu.SemaphoreType.DMA((2,)),
            pltpu.SemaphoreType.DMA,
        ],
        compiler_params=pltpu.CompilerParams(collective_id=0),
    )(pltpu.with_memory_space_constraint(x, pltpu.MemorySpace.HBM))
    return jnp.reshape(out, (m, 1))
